# Initial kernel scaffold; baseline (speedup 1.0000x reference)
import jax
import jax.numpy as jnp
from jax.experimental import pallas as pl


def kernel(end_output, S, emb_in, emb_out, emb3, emb4, mul, bias, means, stds):
    raise NotImplementedError("write your pallas kernel here")


# speedup vs baseline: 9.3745x; 9.3745x over previous
import jax
import jax.numpy as jnp
from jax.experimental import pallas as pl

_BS, _T, _V, _F = 16, 32, 10, 2048
_N = _BS * _T
_ROWS = _BS * _T * _V
_RB = 512
_LUT = 16


def _small_kernel(sl_ref, slt_ref, mul_ref, bias_ref, means_ref, stds_ref,
                  emb3_ref, emb4_ref, ab_ref, d_ref):
    smin = jnp.minimum(sl_ref[...], slt_ref[...])

    d_ref[...] = jnp.sum(smin, axis=0)

    h = bias_ref[...].reshape(_V, _V, 1) * jnp.ones((_V, _V, _N), jnp.float32)
    for m in range(_V):
        h = h + smin[:, m:m + 1, :] * mul_ref[m:m + 1, :].reshape(1, _V, 1)
    mean_j = means_ref[0:1, :].reshape(1, _V, 1)
    std_j = stds_ref[0:1, :].reshape(1, _V, 1)
    a = (2.0 * 3.14159) ** 0.5
    tmp = jnp.exp(-0.5 * ((h - mean_j) / std_j) ** 2) / (a * std_j)
    ef = jnp.tanh(jax.nn.sigmoid(tmp))

    dist = smin
    sp = jnp.zeros((_V, _V, _N), jnp.float32)
    for k in range(_V):
        temp = dist[:, k:k + 1, :] + dist[k:k + 1, :, :]
        new = jnp.minimum(dist, temp)
        x = jnp.where(jnp.equal(new, dist), 0.0, 1.0)
        sp = sp + x * ef
        dist = new

    cnt1 = jnp.sum(dist, axis=1)
    sp_enc = ((_V - cnt1)[:, None, :] * emb3_ref[0:1, :].reshape(1, _V, 1)
              + cnt1[:, None, :] * emb3_ref[1:2, :].reshape(1, _V, 1))

    eidx = sp.astype(jnp.int32)
    ed_enc = jnp.zeros((_V, _V, _N), jnp.float32)
    for dd in range(10):
        w = jnp.sum((eidx == dd).astype(jnp.float32), axis=1)
        ed_enc = ed_enc + w[:, None, :] * emb4_ref[dd:dd + 1, :].reshape(1, _V, 1)

    ab_ref[...] = sp_enc + ed_enc


def _stream_kernel(x_ref, dv_ref, ein_ref, eout_ref, o_ref):
    lut = ein_ref[...] + eout_ref[...]
    dv = dv_ref[...].astype(jnp.int32)
    ids = jax.lax.broadcasted_iota(jnp.int32, (1, _LUT), 1)
    oh = (dv == ids).astype(jnp.float32)
    rows = jax.lax.dot_general(oh, lut, (((1,), (0,)), ((), ())),
                               precision=jax.lax.Precision.HIGHEST,
                               preferred_element_type=jnp.float32)
    o_ref[...] = x_ref[...] + rows


def kernel(end_output, S, emb_in, emb_out, emb3, emb4, mul, bias, means, stds):
    sl = jnp.transpose(S, (2, 3, 0, 1)).reshape(_V, _V, _N)
    slt = jnp.transpose(S, (3, 2, 0, 1)).reshape(_V, _V, _N)

    ab_l, d_l = pl.pallas_call(
        _small_kernel,
        out_shape=(jax.ShapeDtypeStruct((_V, _V, _N), jnp.float32),
                   jax.ShapeDtypeStruct((_V, _N), jnp.float32)),
    )(sl, slt, mul, bias, means, stds, emb3, emb4)

    atten_bias = ab_l.reshape(_V, _V, _BS, _T).transpose(2, 3, 0, 1)
    dv = d_l.transpose(1, 0).reshape(_ROWS, 1)

    out = pl.pallas_call(
        _stream_kernel,
        grid=(_ROWS // _RB,),
        in_specs=[pl.BlockSpec((_RB, _F), lambda i: (i, 0)),
                  pl.BlockSpec((_RB, 1), lambda i: (i, 0)),
                  pl.BlockSpec((_LUT, _F), lambda i: (0, 0)),
                  pl.BlockSpec((_LUT, _F), lambda i: (0, 0))],
        out_specs=pl.BlockSpec((_RB, _F), lambda i: (i, 0)),
        out_shape=jax.ShapeDtypeStruct((_ROWS, _F), jnp.float32),
    )(end_output.reshape(_ROWS, _F), dv, emb_in[:_LUT], emb_out[:_LUT])

    return out.reshape(_BS, _T, _V, _F), atten_bias


# speedup vs baseline: 14.4675x; 1.5433x over previous
import jax
import jax.numpy as jnp
from jax.experimental import pallas as pl

_BS, _T, _V, _F = 16, 32, 10, 2048
_N = _BS * _T
_VV = _V * _V
_LUT = 16
_G = 32


def _mm(a, b):
    return jax.lax.dot_general(a, b, (((1,), (0,)), ((), ())),
                               precision=jax.lax.Precision.HIGHEST,
                               preferred_element_type=jnp.float32)


def _small_kernel(s_ref, m_ref, biasf_ref, meansf_ref, stdsf_ref,
                  e3f_ref, e4f_ref, ab_ref, d_ref):
    s = s_ref[...]
    af = jax.lax.broadcasted_iota(jnp.int32, (_VV, _VV), 0).astype(jnp.float32)
    bf = jax.lax.broadcasted_iota(jnp.int32, (_VV, _VV), 1).astype(jnp.float32)
    bi = jnp.floor(bf * 0.1)
    bj = bf - 10.0 * bi
    ai = jnp.floor(af * 0.1)

    perm = (af == bj * 10.0 + bi).astype(jnp.float32)
    smin = jnp.minimum(s, _mm(s, perm))

    ha = jax.lax.broadcasted_iota(jnp.int32, (_VV, _LUT), 0).astype(jnp.float32)
    hv = jax.lax.broadcasted_iota(jnp.int32, (_VV, _LUT), 1).astype(jnp.float32)
    hsel = ((ha - 10.0 * jnp.floor(ha * 0.1)) == hv).astype(jnp.float32)
    d_ref[...] = _mm(smin, hsel)

    h = _mm(smin, m_ref[...]) + biasf_ref[...]
    a = (2.0 * 3.14159) ** 0.5
    stdsf = stdsf_ref[...]
    tmp = jnp.exp(-0.5 * ((h - meansf_ref[...]) / stdsf) ** 2) / (a * stdsf)
    ef = jnp.tanh(jax.nn.sigmoid(tmp))

    dist = smin
    sp = jnp.zeros((_N, _VV), jnp.float32)
    for k in range(_V):
        ck = ((af == bi * 10.0 + k).astype(jnp.float32)
              + (af == k * 10.0 + bj).astype(jnp.float32))
        temp = _mm(dist, ck)
        new = jnp.minimum(dist, temp)
        x = jnp.where(jnp.equal(new, dist), 0.0, 1.0)
        sp = sp + x * ef
        dist = new

    gsum = (ai == bi).astype(jnp.float32)

    cnt = _mm(dist, gsum)
    sp_enc = (_V - cnt) * e3f_ref[0:1, :] + cnt * e3f_ref[1:2, :]

    eidx = jnp.floor(sp)
    ed_enc = jnp.zeros((_N, _VV), jnp.float32)
    for dd in range(10):
        w = _mm((eidx == float(dd)).astype(jnp.float32), gsum)
        ed_enc = ed_enc + w * e4f_ref[dd:dd + 1, :]

    ab_ref[...] = sp_enc + ed_enc


def _stream_kernel(x_ref, d_ref, ein_ref, eout_ref, o_ref):
    lut = ein_ref[...] + eout_ref[...]
    dv = d_ref[...].astype(jnp.int32)
    ids = jax.lax.broadcasted_iota(jnp.int32, (1, _LUT), 1)
    for v in range(_V):
        oh = (dv[:, v:v + 1] == ids).astype(jnp.float32)
        rows = _mm(oh, lut)
        o_ref[:, v, :] = x_ref[:, v, :] + rows


def kernel(end_output, S, emb_in, emb_out, emb3, emb4, mul, bias, means, stds):
    s2 = S.reshape(_N, _VV)
    m = jnp.kron(jnp.eye(_V, dtype=jnp.float32), mul)
    biasf = bias.reshape(1, _VV)
    meansf = jnp.tile(means, (1, _V))
    stdsf = jnp.tile(stds, (1, _V))
    e3f = jnp.tile(emb3[0:2, :], (1, _V))
    e4f = jnp.tile(emb4, (1, _V))

    ab2, d2 = pl.pallas_call(
        _small_kernel,
        out_shape=(jax.ShapeDtypeStruct((_N, _VV), jnp.float32),
                   jax.ShapeDtypeStruct((_N, _LUT), jnp.float32)),
    )(s2, m, biasf, meansf, stdsf, e3f, e4f)

    out3 = pl.pallas_call(
        _stream_kernel,
        grid=(_N // _G,),
        in_specs=[pl.BlockSpec((_G, _V, _F), lambda i: (i, 0, 0)),
                  pl.BlockSpec((_G, _LUT), lambda i: (i, 0)),
                  pl.BlockSpec((_LUT, _F), lambda i: (0, 0)),
                  pl.BlockSpec((_LUT, _F), lambda i: (0, 0))],
        out_specs=pl.BlockSpec((_G, _V, _F), lambda i: (i, 0, 0)),
        out_shape=jax.ShapeDtypeStruct((_N, _V, _F), jnp.float32),
    )(end_output.reshape(_N, _V, _F), d2, emb_in[:_LUT], emb_out[:_LUT])

    return (out3.reshape(_BS, _T, _V, _F),
            ab2.reshape(_BS, _T, _V, _V))
